# baseline (speedup 1.0000x reference)
import functools

import jax
import jax.numpy as jnp
from jax import lax
from jax.experimental import pallas as pl
from jax.experimental.pallas import tpu as pltpu
from jax.experimental.pallas import tpu_sc as plsc

N = 10000
E = 320000
NC, NS = 2, 16
NW = NC * NS
KW = 128
EPW = 10240
EPAD = NW * EPW
WINS = EPW // KW
NPAD = 10240
RPT = NPAD // NS
BN = 2000
GRID = N // BN


def _mesh():
    return plsc.VectorSubcoreMesh(core_axis_name="c", subcore_axis_name="s",
                                  num_cores=NC, num_subcores=NS)


_SC_PARAMS = pltpu.CompilerParams(use_tc_tiling_on_sc=False)


ZR = 16
RING = 4
AHEAD = 2
IR = 6


def _zero_acc_2d(zb, acc, s, w, zsem):
    zero16 = jnp.zeros((16,), jnp.float32)

    def zfill(i, carry):
        for j in range(w // 16):
            zb[i, pl.ds(j * 16, 16)] = zero16
        return carry

    lax.fori_loop(0, ZR, zfill, 0)
    zds = [pltpu.async_copy(zb, acc.at[pl.ds(s * RPT + t * ZR, ZR)], zsem)
           for t in range(RPT // ZR)]
    for d in zds:
        d.wait()


def _hist(eidx):
    @functools.partial(
        pl.kernel,
        out_type=jax.ShapeDtypeStruct((NC, NPAD), jnp.float32),
        mesh=_mesh(),
        compiler_params=_SC_PARAMS,
        scratch_types=[
            pltpu.VMEM((WINS, 2, KW), jnp.int32),
            pltpu.VMEM((KW,), jnp.float32),
            pltpu.VMEM((RPT,), jnp.float32),
            pltpu.VMEM_SHARED((NPAD,), jnp.float32),
            pltpu.SemaphoreType.DMA,
        ],
    )
    def hist(eidx_hbm, out_hbm, eall, ones_v, zb, acc, ssem):
        c = lax.axis_index("c")
        s = lax.axis_index("s")
        wid = c * NS + s
        ones16 = jnp.ones((16,), jnp.float32)
        zero16 = jnp.zeros((16,), jnp.float32)
        for j in range(KW // 16):
            ones_v[pl.ds(j * 16, 16)] = ones16

        def zfill(i, carry):
            zb[pl.ds(i * 16, 16)] = zero16
            return carry

        lax.fori_loop(0, RPT // 16, zfill, 0)
        pltpu.sync_copy(eidx_hbm.at[wid], eall)
        pltpu.sync_copy(zb, acc.at[pl.ds(s * RPT, RPT)])
        plsc.subcore_barrier()
        G = 8
        for g0 in range(0, WINS, G):
            ds = [pltpu.async_copy(ones_v, acc.at[eall.at[win, 1]], ssem,
                                   add=True)
                  for win in range(g0, g0 + G)]
            for d in ds:
                d.wait()
        plsc.subcore_barrier()
        pltpu.sync_copy(acc.at[pl.ds(s * RPT, RPT)],
                        out_hbm.at[c, pl.ds(s * RPT, RPT)])

    return hist(eidx)


CHK = N // NS


def _edge_scatter_preload(hp, eidx, w):
    @functools.partial(
        pl.kernel,
        out_type=jax.ShapeDtypeStruct((NC, NPAD, w), jnp.float32),
        mesh=_mesh(),
        compiler_params=_SC_PARAMS,
        scratch_types=[
            pltpu.VMEM((WINS, 2, KW), jnp.int32),
            [pltpu.VMEM((KW, w), jnp.float32)] * RING,
            pltpu.VMEM((ZR, w), jnp.float32),
            pltpu.VMEM_SHARED((NPAD, w), jnp.float32),
            [pltpu.SemaphoreType.DMA] * RING,
            [pltpu.SemaphoreType.DMA] * RING,
            pltpu.SemaphoreType.DMA,
        ],
    )
    def scat(hp_hbm, eidx_hbm, out_hbm, eall, rows, zb, acc, gsem, ssem,
             zsem):
        c = lax.axis_index("c")
        s = lax.axis_index("s")
        wid = c * NS + s
        pltpu.sync_copy(eidx_hbm.at[wid], eall)
        _zero_acc_2d(zb, acc, s, w, zsem)
        plsc.subcore_barrier()

        gd = {}
        sd = {}

        def start_gather(win):
            b = win % RING
            gd[win] = pltpu.async_copy(
                hp_hbm.at[eall.at[win, 0]], rows[b], gsem[b])

        for win in range(AHEAD):
            start_gather(win)
        for win in range(WINS):
            b = win % RING
            gd.pop(win).wait()
            sd[win] = pltpu.async_copy(
                rows[b], acc.at[eall.at[win, 1]], ssem[b], add=True)
            nxt = win + AHEAD
            if nxt < WINS:
                prev = nxt - RING
                if prev >= 0:
                    sd.pop(prev).wait()
                start_gather(nxt)
        for win in sorted(sd):
            sd[win].wait()
        plsc.subcore_barrier()
        pltpu.sync_copy(acc.at[pl.ds(s * RPT, RPT)],
                        out_hbm.at[c, pl.ds(s * RPT, RPT)])

    return scat(hp, eidx)


WINS2 = EPAD // NS // KW
HW = 64


def _edge_scatter_colsplit(hp, eidx_cs):
    @functools.partial(
        pl.kernel,
        out_type=jax.ShapeDtypeStruct((NC, NPAD, HW), jnp.float32),
        mesh=_mesh(),
        compiler_params=_SC_PARAMS,
        scratch_types=[
            [pltpu.VMEM((2, KW), jnp.int32)] * IR,
            [pltpu.VMEM((KW, HW), jnp.float32)] * RING,
            pltpu.VMEM((ZR, HW), jnp.float32),
            pltpu.VMEM_SHARED((N, HW), jnp.float32),
            pltpu.VMEM_SHARED((NPAD, HW), jnp.float32),
            [pltpu.SemaphoreType.DMA] * IR,
            [pltpu.SemaphoreType.DMA] * RING,
            [pltpu.SemaphoreType.DMA] * RING,
            pltpu.SemaphoreType.DMA,
        ],
    )
    def scat(hp_hbm, eidx_hbm, out_hbm, ibuf, rows, zb, tbl, acc, isem,
             gsem, ssem, zsem):
        c = lax.axis_index("c")
        s = lax.axis_index("s")
        pltpu.sync_copy(hp_hbm.at[c, pl.ds(s * CHK, CHK)],
                        tbl.at[pl.ds(s * CHK, CHK)])
        idxd = {}
        gd = {}
        sd = {}

        def start_idx(win):
            idxd[win] = pltpu.async_copy(
                eidx_hbm.at[s, win], ibuf[win % IR], isem[win % IR])

        def start_gather(win):
            b = win % RING
            gd[win] = pltpu.async_copy(
                tbl.at[ibuf[win % IR].at[0]], rows[b], gsem[b])

        for win in range(4):
            start_idx(win)
        _zero_acc_2d(zb, acc, s, HW, zsem)
        plsc.subcore_barrier()
        for win in range(AHEAD):
            idxd.pop(win).wait()
            start_gather(win)
        for win in range(WINS2):
            b = win % RING
            gd.pop(win).wait()
            sd[win] = pltpu.async_copy(
                rows[b], acc.at[ibuf[win % IR].at[1]], ssem[b], add=True)
            prev = win - (RING - AHEAD)
            if prev >= 0:
                sd.pop(prev).wait()
            if win + 4 < WINS2:
                start_idx(win + 4)
            nxt = win + AHEAD
            if nxt < WINS2:
                idxd.pop(nxt).wait()
                start_gather(nxt)
        for win in sorted(sd):
            sd[win].wait()
        plsc.subcore_barrier()
        pltpu.sync_copy(acc.at[pl.ds(s * RPT, RPT)],
                        out_hbm.at[c, pl.ds(s * RPT, RPT)])

    return scat(hp, eidx_cs)


def _edge_scatter(hp, eidx, w):
    return _edge_scatter_preload(hp, eidx, w)


_MM = dict(preferred_element_type=jnp.float32,
           precision=jax.lax.Precision.HIGHEST)


def _row_spec(width):
    return pl.BlockSpec((BN, width), lambda i: (i, 0))


def _part_spec(width):
    return pl.BlockSpec((NC, BN, width), lambda i: (0, i, 0))


def _full_spec(shape):
    nd = len(shape)
    return pl.BlockSpec(shape, lambda i: (0,) * nd)


def _dinv_body(dp_ref, o_ref):
    deg = dp_ref[0:80] + dp_ref[80:160] + 1.0
    o_ref[...] = lax.rsqrt(deg)


def _mm1_body(x_ref, w_ref, dv_ref, o_ref, o2_ref):
    v = dv_ref[...] * jnp.dot(x_ref[...], w_ref[...], **_MM)
    o_ref[...] = v
    o2_ref[0] = v[:, :HW]
    o2_ref[1] = v[:, HW:]


def _epmm_body(s_ref, u_ref, dv_ref, b_ref, w_ref, o_ref):
    p = jnp.concatenate([s_ref[0], s_ref[1]], axis=-1) + u_ref[...]
    h = jnp.maximum(dv_ref[...] * p + b_ref[...], 0.0)
    o_ref[...] = dv_ref[...] * jnp.dot(h, w_ref[...], **_MM)


def _mid_body(s_ref, u_ref, dv_ref, b_ref, wmu_ref, bmu_ref, wlv_ref,
              blv_ref, eps_ref, mu_ref, lv_ref, u3_ref):
    p = s_ref[0] + s_ref[1] + u_ref[...]
    h2 = jnp.maximum(dv_ref[...] * p + b_ref[...], 0.0)
    mu = jnp.dot(h2, wmu_ref[...], **_MM) + bmu_ref[...]
    lv = jnp.dot(h2, wlv_ref[...], **_MM) + blv_ref[...]
    z = mu + lv * eps_ref[...]
    mu_ref[...] = mu
    lv_ref[...] = lv
    u3_ref[...] = dv_ref[...] * z


def _aggmm_body(s_ref, u_ref, dv_ref, w_ref, b_ref, o_ref):
    agg = dv_ref[...] * (s_ref[0] + s_ref[1] + u_ref[...])
    h = jnp.maximum(jnp.dot(agg, w_ref[...], **_MM) + b_ref[...], 0.0)
    o_ref[...] = dv_ref[...] * h


def _agg2mm_body(s_ref, u_ref, dv_ref, w4_ref, b4_ref, w5_ref, o_ref,
                 o2_ref):
    agg = dv_ref[...] * (s_ref[0] + s_ref[1] + u_ref[...])
    h4 = jnp.maximum(jnp.dot(agg, w4_ref[...], **_MM) + b4_ref[...], 0.0)
    v = dv_ref[...] * jnp.dot(h4, w5_ref[...], **_MM)
    o_ref[...] = v
    o2_ref[0] = v[:, :HW]
    o2_ref[1] = v[:, HW:]


def _final_body(s_ref, u_ref, dv_ref, b_ref, o_ref):
    p = jnp.concatenate([s_ref[0], s_ref[1]], axis=-1) + u_ref[...]
    o_ref[...] = jax.nn.sigmoid(dv_ref[...] * p + b_ref[...])


def kernel(x, W1, b1, W2, b2, Wmu, bmu, Wlv, blv, W3, b3, W4, b4, W5, b5,
           edge_index):
    f32 = jnp.float32
    src = edge_index[0]
    dst = edge_index[1]
    pad = EPAD - E
    padi = jnp.arange(pad, dtype=jnp.int32)
    srcf = jnp.concatenate([src, padi % N])
    dstf = jnp.concatenate([dst, N + padi % (NPAD - N)])
    srcp = srcf.reshape(NW, WINS, KW)
    dstp = dstf.reshape(NW, WINS, KW)
    eidx = jnp.stack([srcp, dstp], axis=2)
    eidx_cs = jnp.stack([srcf.reshape(NS, WINS2, KW),
                         dstf.reshape(NS, WINS2, KW)], axis=2)

    degp = _hist(eidx)
    dinv80 = pl.pallas_call(
        _dinv_body,
        out_shape=jax.ShapeDtypeStruct((80, 128), f32),
    )(degp.reshape(160, 128))
    dv = dinv80.reshape(NPAD, 1)[:N]

    b1r, b2r, b3r, b4r, b5r = (b.reshape(1, -1) for b in (b1, b2, b3, b4, b5))
    bmur, blvr = bmu.reshape(1, -1), blv.reshape(1, -1)
    eps = jax.random.normal(jax.random.key(1234), (N, Wmu.shape[1]), dtype=f32)

    dv_spec = pl.BlockSpec((BN, 1), lambda i: (i, 0))

    u1, u1s = pl.pallas_call(
        _mm1_body,
        grid=(GRID,),
        in_specs=[_row_spec(128), _full_spec((128, 128)), dv_spec],
        out_specs=[_row_spec(128), _part_spec(HW)],
        out_shape=(jax.ShapeDtypeStruct((N, 128), f32),
                   jax.ShapeDtypeStruct((NC, N, HW), f32)),
    )(x, W1, dv)
    s1 = _edge_scatter_colsplit(u1s, eidx_cs)

    u2 = pl.pallas_call(
        _epmm_body,
        grid=(GRID,),
        in_specs=[_part_spec(HW), _row_spec(128), dv_spec,
                  _full_spec((1, 128)), _full_spec((128, 64))],
        out_specs=_row_spec(64),
        out_shape=jax.ShapeDtypeStruct((N, 64), f32),
    )(s1, u1, dv, b1r, W2)
    s2 = _edge_scatter(u2, eidx, 64)

    mu, lv, u3 = pl.pallas_call(
        _mid_body,
        grid=(GRID,),
        in_specs=[_part_spec(64), _row_spec(64), dv_spec, _full_spec((1, 64)),
                  _full_spec((64, 32)), _full_spec((1, 32)),
                  _full_spec((64, 32)), _full_spec((1, 32)), _row_spec(32)],
        out_specs=[_row_spec(32), _row_spec(32), _row_spec(32)],
        out_shape=(jax.ShapeDtypeStruct((N, 32), f32),
                   jax.ShapeDtypeStruct((N, 32), f32),
                   jax.ShapeDtypeStruct((N, 32), f32)),
    )(s2, u2, dv, b2r, Wmu, bmur, Wlv, blvr, eps)
    s3 = _edge_scatter(u3, eidx, 32)

    u4 = pl.pallas_call(
        _aggmm_body,
        grid=(GRID,),
        in_specs=[_part_spec(32), _row_spec(32), dv_spec,
                  _full_spec((32, 64)), _full_spec((1, 64))],
        out_specs=_row_spec(64),
        out_shape=jax.ShapeDtypeStruct((N, 64), f32),
    )(s3, u3, dv, W3, b3r)
    s4 = _edge_scatter(u4, eidx, 64)

    u5, u5s = pl.pallas_call(
        _agg2mm_body,
        grid=(GRID,),
        in_specs=[_part_spec(64), _row_spec(64), dv_spec,
                  _full_spec((64, 128)), _full_spec((1, 128)),
                  _full_spec((128, 128))],
        out_specs=[_row_spec(128), _part_spec(HW)],
        out_shape=(jax.ShapeDtypeStruct((N, 128), f32),
                   jax.ShapeDtypeStruct((NC, N, HW), f32)),
    )(s4, u4, dv, W4, b4r, W5)
    s5 = _edge_scatter_colsplit(u5s, eidx_cs)

    recon = pl.pallas_call(
        _final_body,
        grid=(GRID,),
        in_specs=[_part_spec(HW), _row_spec(128), dv_spec,
                  _full_spec((1, 128))],
        out_specs=_row_spec(128),
        out_shape=jax.ShapeDtypeStruct((N, 128), f32),
    )(s5, u5, dv, b5r)
    return (recon, mu, lv)


# speedup vs baseline: 1.0327x; 1.0327x over previous
import functools

import jax
import jax.numpy as jnp
from jax import lax
from jax.experimental import pallas as pl
from jax.experimental.pallas import tpu as pltpu
from jax.experimental.pallas import tpu_sc as plsc

N = 10000
E = 320000
NC, NS = 2, 16
NW = NC * NS
KW = 128
EPW = 10240
EPAD = NW * EPW
WINS = EPW // KW
NPAD = 10240
RPT = NPAD // NS
BN = 2000
GRID = N // BN


def _mesh():
    return plsc.VectorSubcoreMesh(core_axis_name="c", subcore_axis_name="s",
                                  num_cores=NC, num_subcores=NS)


_SC_PARAMS = pltpu.CompilerParams(use_tc_tiling_on_sc=False)


ZR = 16
RING = 4
AHEAD = 2
IR = 6


def _zero_acc_2d(zb, acc, s, w, zsem):
    zero16 = jnp.zeros((16,), jnp.float32)

    def zfill(i, carry):
        for j in range(w // 16):
            zb[i, pl.ds(j * 16, 16)] = zero16
        return carry

    lax.fori_loop(0, ZR, zfill, 0)
    zds = [pltpu.async_copy(zb, acc.at[pl.ds(s * RPT + t * ZR, ZR)], zsem)
           for t in range(RPT // ZR)]
    for d in zds:
        d.wait()


def _hist(eidx):
    @functools.partial(
        pl.kernel,
        out_type=jax.ShapeDtypeStruct((NC, NPAD), jnp.float32),
        mesh=_mesh(),
        compiler_params=_SC_PARAMS,
        scratch_types=[
            pltpu.VMEM((WINS, 2, KW), jnp.int32),
            pltpu.VMEM((KW,), jnp.float32),
            pltpu.VMEM((RPT,), jnp.float32),
            pltpu.VMEM_SHARED((NPAD,), jnp.float32),
            pltpu.SemaphoreType.DMA,
        ],
    )
    def hist(eidx_hbm, out_hbm, eall, ones_v, zb, acc, ssem):
        c = lax.axis_index("c")
        s = lax.axis_index("s")
        wid = c * NS + s
        ones16 = jnp.ones((16,), jnp.float32)
        zero16 = jnp.zeros((16,), jnp.float32)
        for j in range(KW // 16):
            ones_v[pl.ds(j * 16, 16)] = ones16

        def zfill(i, carry):
            zb[pl.ds(i * 16, 16)] = zero16
            return carry

        lax.fori_loop(0, RPT // 16, zfill, 0)
        pltpu.sync_copy(eidx_hbm.at[wid], eall)
        pltpu.sync_copy(zb, acc.at[pl.ds(s * RPT, RPT)])
        plsc.subcore_barrier()
        G = 8
        for g0 in range(0, WINS, G):
            ds = [pltpu.async_copy(ones_v, acc.at[eall.at[win, 1]], ssem,
                                   add=True)
                  for win in range(g0, g0 + G)]
            for d in ds:
                d.wait()
        plsc.subcore_barrier()
        pltpu.sync_copy(acc.at[pl.ds(s * RPT, RPT)],
                        out_hbm.at[c, pl.ds(s * RPT, RPT)])

    return hist(eidx)


CHK = N // NS


def _edge_scatter_preload(hp, eidx, w):
    @functools.partial(
        pl.kernel,
        out_type=jax.ShapeDtypeStruct((NC, NPAD, w), jnp.float32),
        mesh=_mesh(),
        compiler_params=_SC_PARAMS,
        scratch_types=[
            pltpu.VMEM((WINS, 2, KW), jnp.int32),
            [pltpu.VMEM((KW, w), jnp.float32)] * RING,
            pltpu.VMEM((ZR, w), jnp.float32),
            pltpu.VMEM_SHARED((NPAD, w), jnp.float32),
            [pltpu.SemaphoreType.DMA] * RING,
            [pltpu.SemaphoreType.DMA] * RING,
            pltpu.SemaphoreType.DMA,
        ],
    )
    def scat(hp_hbm, eidx_hbm, out_hbm, eall, rows, zb, acc, gsem, ssem,
             zsem):
        c = lax.axis_index("c")
        s = lax.axis_index("s")
        wid = c * NS + s
        pltpu.sync_copy(eidx_hbm.at[wid], eall)
        _zero_acc_2d(zb, acc, s, w, zsem)
        plsc.subcore_barrier()

        gd = {}
        sd = {}

        def start_gather(win):
            b = win % RING
            gd[win] = pltpu.async_copy(
                hp_hbm.at[eall.at[win, 0]], rows[b], gsem[b])

        for win in range(AHEAD):
            start_gather(win)
        for win in range(WINS):
            b = win % RING
            gd.pop(win).wait()
            sd[win] = pltpu.async_copy(
                rows[b], acc.at[eall.at[win, 1]], ssem[b], add=True)
            nxt = win + AHEAD
            if nxt < WINS:
                prev = nxt - RING
                if prev >= 0:
                    sd.pop(prev).wait()
                start_gather(nxt)
        for win in sorted(sd):
            sd[win].wait()
        plsc.subcore_barrier()
        pltpu.sync_copy(acc.at[pl.ds(s * RPT, RPT)],
                        out_hbm.at[c, pl.ds(s * RPT, RPT)])

    return scat(hp, eidx)


WINS2 = EPAD // NS // KW
HW = 64


def _edge_scatter_colsplit(hp, eidx_cs):
    @functools.partial(
        pl.kernel,
        out_type=jax.ShapeDtypeStruct((NC, NPAD, HW), jnp.float32),
        mesh=_mesh(),
        compiler_params=_SC_PARAMS,
        scratch_types=[
            [pltpu.VMEM((2, KW), jnp.int32)] * IR,
            [pltpu.VMEM((KW, HW), jnp.float32)] * RING,
            pltpu.VMEM((ZR, HW), jnp.float32),
            pltpu.VMEM_SHARED((N, HW), jnp.float32),
            pltpu.VMEM_SHARED((NPAD, HW), jnp.float32),
            [pltpu.SemaphoreType.DMA] * IR,
            [pltpu.SemaphoreType.DMA] * RING,
            [pltpu.SemaphoreType.DMA] * RING,
            pltpu.SemaphoreType.DMA,
        ],
    )
    def scat(hp_hbm, eidx_hbm, out_hbm, ibuf, rows, zb, tbl, acc, isem,
             gsem, ssem, zsem):
        c = lax.axis_index("c")
        s = lax.axis_index("s")
        pltpu.sync_copy(hp_hbm.at[pl.ds(s * CHK, CHK), pl.ds(c * HW, HW)],
                        tbl.at[pl.ds(s * CHK, CHK)])
        idxd = {}
        gd = {}
        sd = {}

        def start_idx(win):
            idxd[win] = pltpu.async_copy(
                eidx_hbm.at[s, win], ibuf[win % IR], isem[win % IR])

        def start_gather(win):
            b = win % RING
            gd[win] = pltpu.async_copy(
                tbl.at[ibuf[win % IR].at[0]], rows[b], gsem[b])

        for win in range(4):
            start_idx(win)
        _zero_acc_2d(zb, acc, s, HW, zsem)
        plsc.subcore_barrier()
        for win in range(AHEAD):
            idxd.pop(win).wait()
            start_gather(win)
        for win in range(WINS2):
            b = win % RING
            gd.pop(win).wait()
            sd[win] = pltpu.async_copy(
                rows[b], acc.at[ibuf[win % IR].at[1]], ssem[b], add=True)
            prev = win - (RING - AHEAD)
            if prev >= 0:
                sd.pop(prev).wait()
            if win + 4 < WINS2:
                start_idx(win + 4)
            nxt = win + AHEAD
            if nxt < WINS2:
                idxd.pop(nxt).wait()
                start_gather(nxt)
        for win in sorted(sd):
            sd[win].wait()
        plsc.subcore_barrier()
        pltpu.sync_copy(acc.at[pl.ds(s * RPT, RPT)],
                        out_hbm.at[c, pl.ds(s * RPT, RPT)])

    return scat(hp, eidx_cs)


def _edge_scatter(hp, eidx, w):
    return _edge_scatter_preload(hp, eidx, w)


_MM = dict(preferred_element_type=jnp.float32,
           precision=jax.lax.Precision.HIGHEST)


def _row_spec(width):
    return pl.BlockSpec((BN, width), lambda i: (i, 0))


def _part_spec(width):
    return pl.BlockSpec((NC, BN, width), lambda i: (0, i, 0))


def _full_spec(shape):
    nd = len(shape)
    return pl.BlockSpec(shape, lambda i: (0,) * nd)


def _dinv_body(dp_ref, o_ref):
    deg = dp_ref[0:80] + dp_ref[80:160] + 1.0
    o_ref[...] = lax.rsqrt(deg)


def _mm1_body(x_ref, w_ref, dv_ref, o_ref):
    o_ref[...] = dv_ref[...] * jnp.dot(x_ref[...], w_ref[...], **_MM)


def _epmm_body(s_ref, u_ref, dv_ref, b_ref, w_ref, o_ref):
    p = jnp.concatenate([s_ref[0], s_ref[1]], axis=-1) + u_ref[...]
    h = jnp.maximum(dv_ref[...] * p + b_ref[...], 0.0)
    o_ref[...] = dv_ref[...] * jnp.dot(h, w_ref[...], **_MM)


def _mid_body(s_ref, u_ref, dv_ref, b_ref, wmu_ref, bmu_ref, wlv_ref,
              blv_ref, eps_ref, mu_ref, lv_ref, u3_ref):
    p = s_ref[0] + s_ref[1] + u_ref[...]
    h2 = jnp.maximum(dv_ref[...] * p + b_ref[...], 0.0)
    mu = jnp.dot(h2, wmu_ref[...], **_MM) + bmu_ref[...]
    lv = jnp.dot(h2, wlv_ref[...], **_MM) + blv_ref[...]
    z = mu + lv * eps_ref[...]
    mu_ref[...] = mu
    lv_ref[...] = lv
    u3_ref[...] = dv_ref[...] * z


def _aggmm_body(s_ref, u_ref, dv_ref, w_ref, b_ref, o_ref):
    agg = dv_ref[...] * (s_ref[0] + s_ref[1] + u_ref[...])
    h = jnp.maximum(jnp.dot(agg, w_ref[...], **_MM) + b_ref[...], 0.0)
    o_ref[...] = dv_ref[...] * h


def _agg2mm_body(s_ref, u_ref, dv_ref, w4_ref, b4_ref, w5_ref, o_ref):
    agg = dv_ref[...] * (s_ref[0] + s_ref[1] + u_ref[...])
    h4 = jnp.maximum(jnp.dot(agg, w4_ref[...], **_MM) + b4_ref[...], 0.0)
    o_ref[...] = dv_ref[...] * jnp.dot(h4, w5_ref[...], **_MM)


def _final_body(s_ref, u_ref, dv_ref, b_ref, o_ref):
    p = jnp.concatenate([s_ref[0], s_ref[1]], axis=-1) + u_ref[...]
    o_ref[...] = jax.nn.sigmoid(dv_ref[...] * p + b_ref[...])


def kernel(x, W1, b1, W2, b2, Wmu, bmu, Wlv, blv, W3, b3, W4, b4, W5, b5,
           edge_index):
    f32 = jnp.float32
    src = edge_index[0]
    dst = edge_index[1]
    pad = EPAD - E
    padi = jnp.arange(pad, dtype=jnp.int32)
    srcf = jnp.concatenate([src, padi % N])
    dstf = jnp.concatenate([dst, N + padi % (NPAD - N)])
    srcp = srcf.reshape(NW, WINS, KW)
    dstp = dstf.reshape(NW, WINS, KW)
    eidx = jnp.stack([srcp, dstp], axis=2)
    eidx_cs = jnp.stack([srcf.reshape(NS, WINS2, KW),
                         dstf.reshape(NS, WINS2, KW)], axis=2)

    degp = _hist(eidx)
    dinv80 = pl.pallas_call(
        _dinv_body,
        out_shape=jax.ShapeDtypeStruct((80, 128), f32),
    )(degp.reshape(160, 128))
    dv = dinv80.reshape(NPAD, 1)[:N]

    b1r, b2r, b3r, b4r, b5r = (b.reshape(1, -1) for b in (b1, b2, b3, b4, b5))
    bmur, blvr = bmu.reshape(1, -1), blv.reshape(1, -1)
    eps = jax.random.normal(jax.random.key(1234), (N, Wmu.shape[1]), dtype=f32)

    dv_spec = pl.BlockSpec((BN, 1), lambda i: (i, 0))

    u1 = pl.pallas_call(
        _mm1_body,
        grid=(GRID,),
        in_specs=[_row_spec(128), _full_spec((128, 128)), dv_spec],
        out_specs=_row_spec(128),
        out_shape=jax.ShapeDtypeStruct((N, 128), f32),
    )(x, W1, dv)
    s1 = _edge_scatter_colsplit(u1, eidx_cs)

    u2 = pl.pallas_call(
        _epmm_body,
        grid=(GRID,),
        in_specs=[_part_spec(HW), _row_spec(128), dv_spec,
                  _full_spec((1, 128)), _full_spec((128, 64))],
        out_specs=_row_spec(64),
        out_shape=jax.ShapeDtypeStruct((N, 64), f32),
    )(s1, u1, dv, b1r, W2)
    s2 = _edge_scatter(u2, eidx, 64)

    mu, lv, u3 = pl.pallas_call(
        _mid_body,
        grid=(GRID,),
        in_specs=[_part_spec(64), _row_spec(64), dv_spec, _full_spec((1, 64)),
                  _full_spec((64, 32)), _full_spec((1, 32)),
                  _full_spec((64, 32)), _full_spec((1, 32)), _row_spec(32)],
        out_specs=[_row_spec(32), _row_spec(32), _row_spec(32)],
        out_shape=(jax.ShapeDtypeStruct((N, 32), f32),
                   jax.ShapeDtypeStruct((N, 32), f32),
                   jax.ShapeDtypeStruct((N, 32), f32)),
    )(s2, u2, dv, b2r, Wmu, bmur, Wlv, blvr, eps)
    s3 = _edge_scatter(u3, eidx, 32)

    u4 = pl.pallas_call(
        _aggmm_body,
        grid=(GRID,),
        in_specs=[_part_spec(32), _row_spec(32), dv_spec,
                  _full_spec((32, 64)), _full_spec((1, 64))],
        out_specs=_row_spec(64),
        out_shape=jax.ShapeDtypeStruct((N, 64), f32),
    )(s3, u3, dv, W3, b3r)
    s4 = _edge_scatter(u4, eidx, 64)

    u5 = pl.pallas_call(
        _agg2mm_body,
        grid=(GRID,),
        in_specs=[_part_spec(64), _row_spec(64), dv_spec,
                  _full_spec((64, 128)), _full_spec((1, 128)),
                  _full_spec((128, 128))],
        out_specs=_row_spec(128),
        out_shape=jax.ShapeDtypeStruct((N, 128), f32),
    )(s4, u4, dv, W4, b4r, W5)
    s5 = _edge_scatter_colsplit(u5, eidx_cs)

    recon = pl.pallas_call(
        _final_body,
        grid=(GRID,),
        in_specs=[_part_spec(HW), _row_spec(128), dv_spec,
                  _full_spec((1, 128))],
        out_specs=_row_spec(128),
        out_shape=jax.ShapeDtypeStruct((N, 128), f32),
    )(s5, u5, dv, b5r)
    return (recon, mu, lv)


# speedup vs baseline: 1.1031x; 1.0681x over previous
import functools

import jax
import jax.numpy as jnp
from jax import lax
from jax.experimental import pallas as pl
from jax.experimental.pallas import tpu as pltpu
from jax.experimental.pallas import tpu_sc as plsc

N = 10000
E = 320000
NC, NS = 2, 16
NW = NC * NS
KW = 128
EPW = 10240
EPAD = NW * EPW
WINS = EPW // KW
NPAD = 10240
RPT = NPAD // NS
BN = 2000
GRID = N // BN


def _mesh():
    return plsc.VectorSubcoreMesh(core_axis_name="c", subcore_axis_name="s",
                                  num_cores=NC, num_subcores=NS)


_SC_PARAMS = pltpu.CompilerParams(use_tc_tiling_on_sc=False)


ZR = 16
RING = 4
AHEAD = 2
IR = 6


def _zero_acc_2d(zb, acc, s, w, zsem):
    zero16 = jnp.zeros((16,), jnp.float32)

    def zfill(i, carry):
        for j in range(w // 16):
            zb[i, pl.ds(j * 16, 16)] = zero16
        return carry

    lax.fori_loop(0, ZR, zfill, 0)
    zds = [pltpu.async_copy(zb, acc.at[pl.ds(s * RPT + t * ZR, ZR)], zsem)
           for t in range(RPT // ZR)]
    for d in zds:
        d.wait()


def _hist(eidx):
    @functools.partial(
        pl.kernel,
        out_type=jax.ShapeDtypeStruct((NC, NPAD), jnp.float32),
        mesh=_mesh(),
        compiler_params=_SC_PARAMS,
        scratch_types=[
            pltpu.VMEM((WINS, 2, KW), jnp.int32),
            pltpu.VMEM((KW,), jnp.float32),
            pltpu.VMEM((RPT,), jnp.float32),
            pltpu.VMEM_SHARED((NPAD,), jnp.float32),
            pltpu.SemaphoreType.DMA,
        ],
    )
    def hist(eidx_hbm, out_hbm, eall, ones_v, zb, acc, ssem):
        c = lax.axis_index("c")
        s = lax.axis_index("s")
        wid = c * NS + s
        ones16 = jnp.ones((16,), jnp.float32)
        zero16 = jnp.zeros((16,), jnp.float32)
        for j in range(KW // 16):
            ones_v[pl.ds(j * 16, 16)] = ones16

        def zfill(i, carry):
            zb[pl.ds(i * 16, 16)] = zero16
            return carry

        lax.fori_loop(0, RPT // 16, zfill, 0)
        pltpu.sync_copy(eidx_hbm.at[wid], eall)
        pltpu.sync_copy(zb, acc.at[pl.ds(s * RPT, RPT)])
        plsc.subcore_barrier()
        G = 8
        for g0 in range(0, WINS, G):
            ds = [pltpu.async_copy(ones_v, acc.at[eall.at[win, 1]], ssem,
                                   add=True)
                  for win in range(g0, g0 + G)]
            for d in ds:
                d.wait()
        plsc.subcore_barrier()
        pltpu.sync_copy(acc.at[pl.ds(s * RPT, RPT)],
                        out_hbm.at[c, pl.ds(s * RPT, RPT)])

    return hist(eidx)


CHK = N // NS
CLIP = N - (NS - 1) * RPT


def _drain_cols(acc, out_hbm, s, c, w):
    pltpu.sync_copy(acc.at[pl.ds(s * RPT, CLIP)],
                    out_hbm.at[pl.ds(s * RPT, CLIP), pl.ds(c * w, w)])

    @pl.when(s < NS - 1)
    def _():
        pltpu.sync_copy(
            acc.at[pl.ds(s * RPT + CLIP, RPT - CLIP)],
            out_hbm.at[pl.ds(s * RPT + CLIP, RPT - CLIP), pl.ds(c * w, w)])


def _edge_scatter_preload(hp, eidx, w):
    merged = (w == 64)
    oty = (jax.ShapeDtypeStruct((N, NC * w), jnp.float32) if merged
           else jax.ShapeDtypeStruct((NC, NPAD, w), jnp.float32))

    @functools.partial(
        pl.kernel,
        out_type=oty,
        mesh=_mesh(),
        compiler_params=_SC_PARAMS,
        scratch_types=[
            pltpu.VMEM((WINS, 2, KW), jnp.int32),
            [pltpu.VMEM((KW, w), jnp.float32)] * RING,
            pltpu.VMEM((ZR, w), jnp.float32),
            pltpu.VMEM_SHARED((NPAD, w), jnp.float32),
            [pltpu.SemaphoreType.DMA] * RING,
            [pltpu.SemaphoreType.DMA] * RING,
            pltpu.SemaphoreType.DMA,
        ],
    )
    def scat(hp_hbm, eidx_hbm, out_hbm, eall, rows, zb, acc, gsem, ssem,
             zsem):
        c = lax.axis_index("c")
        s = lax.axis_index("s")
        wid = c * NS + s
        pltpu.sync_copy(eidx_hbm.at[wid], eall)
        _zero_acc_2d(zb, acc, s, w, zsem)
        plsc.subcore_barrier()

        gd = {}
        sd = {}

        def start_gather(win):
            b = win % RING
            gd[win] = pltpu.async_copy(
                hp_hbm.at[eall.at[win, 0]], rows[b], gsem[b])

        for win in range(AHEAD):
            start_gather(win)
        for win in range(WINS):
            b = win % RING
            gd.pop(win).wait()
            sd[win] = pltpu.async_copy(
                rows[b], acc.at[eall.at[win, 1]], ssem[b], add=True)
            nxt = win + AHEAD
            if nxt < WINS:
                prev = nxt - RING
                if prev >= 0:
                    sd.pop(prev).wait()
                start_gather(nxt)
        for win in sorted(sd):
            sd[win].wait()
        plsc.subcore_barrier()
        if merged:
            _drain_cols(acc, out_hbm, s, c, w)
        else:
            pltpu.sync_copy(acc.at[pl.ds(s * RPT, RPT)],
                            out_hbm.at[c, pl.ds(s * RPT, RPT)])

    return scat(hp, eidx)


WINS2 = EPAD // NS // KW
HW = 64


def _edge_scatter_colsplit(hp, eidx_cs):
    @functools.partial(
        pl.kernel,
        out_type=jax.ShapeDtypeStruct((N, NC * HW), jnp.float32),
        mesh=_mesh(),
        compiler_params=_SC_PARAMS,
        scratch_types=[
            [pltpu.VMEM((2, KW), jnp.int32)] * IR,
            [pltpu.VMEM((KW, HW), jnp.float32)] * RING,
            pltpu.VMEM((ZR, HW), jnp.float32),
            pltpu.VMEM_SHARED((N, HW), jnp.float32),
            pltpu.VMEM_SHARED((NPAD, HW), jnp.float32),
            [pltpu.SemaphoreType.DMA] * IR,
            [pltpu.SemaphoreType.DMA] * RING,
            [pltpu.SemaphoreType.DMA] * RING,
            pltpu.SemaphoreType.DMA,
        ],
    )
    def scat(hp_hbm, eidx_hbm, out_hbm, ibuf, rows, zb, tbl, acc, isem,
             gsem, ssem, zsem):
        c = lax.axis_index("c")
        s = lax.axis_index("s")
        pltpu.sync_copy(hp_hbm.at[pl.ds(s * CHK, CHK), pl.ds(c * HW, HW)],
                        tbl.at[pl.ds(s * CHK, CHK)])
        idxd = {}
        gd = {}
        sd = {}

        def start_idx(win):
            idxd[win] = pltpu.async_copy(
                eidx_hbm.at[s, win], ibuf[win % IR], isem[win % IR])

        def start_gather(win):
            b = win % RING
            gd[win] = pltpu.async_copy(
                tbl.at[ibuf[win % IR].at[0]], rows[b], gsem[b])

        for win in range(4):
            start_idx(win)
        _zero_acc_2d(zb, acc, s, HW, zsem)
        plsc.subcore_barrier()
        for win in range(AHEAD):
            idxd.pop(win).wait()
            start_gather(win)
        for win in range(WINS2):
            b = win % RING
            gd.pop(win).wait()
            sd[win] = pltpu.async_copy(
                rows[b], acc.at[ibuf[win % IR].at[1]], ssem[b], add=True)
            prev = win - (RING - AHEAD)
            if prev >= 0:
                sd.pop(prev).wait()
            if win + 4 < WINS2:
                start_idx(win + 4)
            nxt = win + AHEAD
            if nxt < WINS2:
                idxd.pop(nxt).wait()
                start_gather(nxt)
        for win in sorted(sd):
            sd[win].wait()
        plsc.subcore_barrier()
        _drain_cols(acc, out_hbm, s, c, HW)

    return scat(hp, eidx_cs)


def _edge_scatter(hp, eidx, w):
    return _edge_scatter_preload(hp, eidx, w)


_MM = dict(preferred_element_type=jnp.float32,
           precision=jax.lax.Precision.HIGHEST)


def _row_spec(width):
    return pl.BlockSpec((BN, width), lambda i: (i, 0))


def _part_spec(width):
    return pl.BlockSpec((NC, BN, width), lambda i: (0, i, 0))


def _full_spec(shape):
    nd = len(shape)
    return pl.BlockSpec(shape, lambda i: (0,) * nd)


def _dinv_body(dp_ref, o_ref):
    deg = dp_ref[0:80] + dp_ref[80:160] + 1.0
    o_ref[...] = lax.rsqrt(deg)


def _mm1_body(x_ref, w_ref, dv_ref, o_ref):
    o_ref[...] = dv_ref[...] * jnp.dot(x_ref[...], w_ref[...], **_MM)


def _epmm_body(s_ref, u_ref, dv_ref, b_ref, w_ref, o_ref):
    p = s_ref[...] + u_ref[...]
    h = jnp.maximum(dv_ref[...] * p + b_ref[...], 0.0)
    o_ref[...] = dv_ref[...] * jnp.dot(h, w_ref[...], **_MM)


def _mid_body(s_ref, u_ref, dv_ref, b_ref, wmu_ref, bmu_ref, wlv_ref,
              blv_ref, eps_ref, mu_ref, lv_ref, u3_ref):
    p = s_ref[:, :64] + s_ref[:, 64:] + u_ref[...]
    h2 = jnp.maximum(dv_ref[...] * p + b_ref[...], 0.0)
    mu = jnp.dot(h2, wmu_ref[...], **_MM) + bmu_ref[...]
    lv = jnp.dot(h2, wlv_ref[...], **_MM) + blv_ref[...]
    z = mu + lv * eps_ref[...]
    mu_ref[...] = mu
    lv_ref[...] = lv
    u3_ref[...] = dv_ref[...] * z


def _aggmm_body(s_ref, u_ref, dv_ref, w_ref, b_ref, o_ref):
    agg = dv_ref[...] * (s_ref[0] + s_ref[1] + u_ref[...])
    h = jnp.maximum(jnp.dot(agg, w_ref[...], **_MM) + b_ref[...], 0.0)
    o_ref[...] = dv_ref[...] * h


def _agg2mm_body(s_ref, u_ref, dv_ref, w4_ref, b4_ref, w5_ref, o_ref):
    agg = dv_ref[...] * (s_ref[:, :64] + s_ref[:, 64:] + u_ref[...])
    h4 = jnp.maximum(jnp.dot(agg, w4_ref[...], **_MM) + b4_ref[...], 0.0)
    o_ref[...] = dv_ref[...] * jnp.dot(h4, w5_ref[...], **_MM)


def _final_body(s_ref, u_ref, dv_ref, b_ref, o_ref):
    p = s_ref[...] + u_ref[...]
    o_ref[...] = jax.nn.sigmoid(dv_ref[...] * p + b_ref[...])


def kernel(x, W1, b1, W2, b2, Wmu, bmu, Wlv, blv, W3, b3, W4, b4, W5, b5,
           edge_index):
    f32 = jnp.float32
    src = edge_index[0]
    dst = edge_index[1]
    pad = EPAD - E
    padi = jnp.arange(pad, dtype=jnp.int32)
    srcf = jnp.concatenate([src, padi % N])
    dstf = jnp.concatenate([dst, N + padi % (NPAD - N)])
    srcp = srcf.reshape(NW, WINS, KW)
    dstp = dstf.reshape(NW, WINS, KW)
    eidx = jnp.stack([srcp, dstp], axis=2)
    eidx_cs = jnp.stack([srcf.reshape(NS, WINS2, KW),
                         dstf.reshape(NS, WINS2, KW)], axis=2)

    degp = _hist(eidx)
    dinv80 = pl.pallas_call(
        _dinv_body,
        out_shape=jax.ShapeDtypeStruct((80, 128), f32),
    )(degp.reshape(160, 128))
    dv = dinv80.reshape(NPAD, 1)[:N]

    b1r, b2r, b3r, b4r, b5r = (b.reshape(1, -1) for b in (b1, b2, b3, b4, b5))
    bmur, blvr = bmu.reshape(1, -1), blv.reshape(1, -1)
    eps = jax.random.normal(jax.random.key(1234), (N, Wmu.shape[1]), dtype=f32)

    dv_spec = pl.BlockSpec((BN, 1), lambda i: (i, 0))

    u1 = pl.pallas_call(
        _mm1_body,
        grid=(GRID,),
        in_specs=[_row_spec(128), _full_spec((128, 128)), dv_spec],
        out_specs=_row_spec(128),
        out_shape=jax.ShapeDtypeStruct((N, 128), f32),
    )(x, W1, dv)
    s1 = _edge_scatter_colsplit(u1, eidx_cs)

    u2 = pl.pallas_call(
        _epmm_body,
        grid=(GRID,),
        in_specs=[_row_spec(128), _row_spec(128), dv_spec,
                  _full_spec((1, 128)), _full_spec((128, 64))],
        out_specs=_row_spec(64),
        out_shape=jax.ShapeDtypeStruct((N, 64), f32),
    )(s1, u1, dv, b1r, W2)
    s2 = _edge_scatter(u2, eidx, 64)

    mu, lv, u3 = pl.pallas_call(
        _mid_body,
        grid=(GRID,),
        in_specs=[_row_spec(128), _row_spec(64), dv_spec, _full_spec((1, 64)),
                  _full_spec((64, 32)), _full_spec((1, 32)),
                  _full_spec((64, 32)), _full_spec((1, 32)), _row_spec(32)],
        out_specs=[_row_spec(32), _row_spec(32), _row_spec(32)],
        out_shape=(jax.ShapeDtypeStruct((N, 32), f32),
                   jax.ShapeDtypeStruct((N, 32), f32),
                   jax.ShapeDtypeStruct((N, 32), f32)),
    )(s2, u2, dv, b2r, Wmu, bmur, Wlv, blvr, eps)
    s3 = _edge_scatter(u3, eidx, 32)

    u4 = pl.pallas_call(
        _aggmm_body,
        grid=(GRID,),
        in_specs=[_part_spec(32), _row_spec(32), dv_spec,
                  _full_spec((32, 64)), _full_spec((1, 64))],
        out_specs=_row_spec(64),
        out_shape=jax.ShapeDtypeStruct((N, 64), f32),
    )(s3, u3, dv, W3, b3r)
    s4 = _edge_scatter(u4, eidx, 64)

    u5 = pl.pallas_call(
        _agg2mm_body,
        grid=(GRID,),
        in_specs=[_row_spec(128), _row_spec(64), dv_spec,
                  _full_spec((64, 128)), _full_spec((1, 128)),
                  _full_spec((128, 128))],
        out_specs=_row_spec(128),
        out_shape=jax.ShapeDtypeStruct((N, 128), f32),
    )(s4, u4, dv, W4, b4r, W5)
    s5 = _edge_scatter_colsplit(u5, eidx_cs)

    recon = pl.pallas_call(
        _final_body,
        grid=(GRID,),
        in_specs=[_row_spec(128), _row_spec(128), dv_spec,
                  _full_spec((1, 128))],
        out_specs=_row_spec(128),
        out_shape=jax.ShapeDtypeStruct((N, 128), f32),
    )(s5, u5, dv, b5r)
    return (recon, mu, lv)


# speedup vs baseline: 1.1117x; 1.0078x over previous
import functools

import jax
import jax.numpy as jnp
from jax import lax
from jax.experimental import pallas as pl
from jax.experimental.pallas import tpu as pltpu
from jax.experimental.pallas import tpu_sc as plsc

N = 10000
E = 320000
NC, NS = 2, 16
NW = NC * NS
KW = 128
EPW = 10240
EPAD = NW * EPW
WINS = EPW // KW
NPAD = 10240
RPT = NPAD // NS
BN = 2000
GRID = N // BN


def _mesh():
    return plsc.VectorSubcoreMesh(core_axis_name="c", subcore_axis_name="s",
                                  num_cores=NC, num_subcores=NS)


_SC_PARAMS = pltpu.CompilerParams(use_tc_tiling_on_sc=False)


ZR = 16
RING = 4
AHEAD = 2
IR = 6


def _zero_acc_2d(zb, acc, s, w, zsem):
    zero16 = jnp.zeros((16,), jnp.float32)

    def zfill(i, carry):
        for j in range(w // 16):
            zb[i, pl.ds(j * 16, 16)] = zero16
        return carry

    lax.fori_loop(0, ZR, zfill, 0)
    zds = [pltpu.async_copy(zb, acc.at[pl.ds(s * RPT + t * ZR, ZR)], zsem)
           for t in range(RPT // ZR)]
    for d in zds:
        d.wait()


def _hist(eidx):
    @functools.partial(
        pl.kernel,
        out_type=jax.ShapeDtypeStruct((NC, NPAD), jnp.float32),
        mesh=_mesh(),
        compiler_params=_SC_PARAMS,
        scratch_types=[
            pltpu.VMEM((WINS, 2, KW), jnp.int32),
            pltpu.VMEM((KW,), jnp.float32),
            pltpu.VMEM((RPT,), jnp.float32),
            pltpu.VMEM_SHARED((NPAD,), jnp.float32),
            pltpu.SemaphoreType.DMA,
        ],
    )
    def hist(eidx_hbm, out_hbm, eall, ones_v, zb, acc, ssem):
        c = lax.axis_index("c")
        s = lax.axis_index("s")
        wid = c * NS + s
        ones16 = jnp.ones((16,), jnp.float32)
        zero16 = jnp.zeros((16,), jnp.float32)
        for j in range(KW // 16):
            ones_v[pl.ds(j * 16, 16)] = ones16

        def zfill(i, carry):
            zb[pl.ds(i * 16, 16)] = zero16
            return carry

        lax.fori_loop(0, RPT // 16, zfill, 0)
        pltpu.sync_copy(eidx_hbm.at[wid], eall)
        pltpu.sync_copy(zb, acc.at[pl.ds(s * RPT, RPT)])
        plsc.subcore_barrier()
        G = 8
        for g0 in range(0, WINS, G):
            ds = [pltpu.async_copy(ones_v, acc.at[eall.at[win, 1]], ssem,
                                   add=True)
                  for win in range(g0, g0 + G)]
            for d in ds:
                d.wait()
        plsc.subcore_barrier()
        pltpu.sync_copy(acc.at[pl.ds(s * RPT, RPT)],
                        out_hbm.at[c, pl.ds(s * RPT, RPT)])

    return hist(eidx)


CHK = N // NS
CLIP = N - (NS - 1) * RPT


def _drain_cols(acc, out_hbm, s, c, w):
    pltpu.sync_copy(acc.at[pl.ds(s * RPT, CLIP)],
                    out_hbm.at[pl.ds(s * RPT, CLIP), pl.ds(c * w, w)])

    @pl.when(s < NS - 1)
    def _():
        pltpu.sync_copy(
            acc.at[pl.ds(s * RPT + CLIP, RPT - CLIP)],
            out_hbm.at[pl.ds(s * RPT + CLIP, RPT - CLIP), pl.ds(c * w, w)])


def _edge_scatter_preload(hp, eidx, w):
    merged = (w == 64)
    oty = (jax.ShapeDtypeStruct((N, NC * w), jnp.float32) if merged
           else jax.ShapeDtypeStruct((NC, NPAD, w), jnp.float32))

    @functools.partial(
        pl.kernel,
        out_type=oty,
        mesh=_mesh(),
        compiler_params=_SC_PARAMS,
        scratch_types=[
            pltpu.VMEM((WINS, 2, KW), jnp.int32),
            [pltpu.VMEM((KW, w), jnp.float32)] * RING,
            pltpu.VMEM((ZR, w), jnp.float32),
            pltpu.VMEM_SHARED((NPAD, w), jnp.float32),
            [pltpu.SemaphoreType.DMA] * RING,
            [pltpu.SemaphoreType.DMA] * RING,
            pltpu.SemaphoreType.DMA,
        ],
    )
    def scat(hp_hbm, eidx_hbm, out_hbm, eall, rows, zb, acc, gsem, ssem,
             zsem):
        c = lax.axis_index("c")
        s = lax.axis_index("s")
        wid = c * NS + s
        pltpu.sync_copy(eidx_hbm.at[wid], eall)
        _zero_acc_2d(zb, acc, s, w, zsem)
        plsc.subcore_barrier()

        gd = {}
        sd = {}

        def start_gather(win):
            b = win % RING
            gd[win] = pltpu.async_copy(
                hp_hbm.at[eall.at[win, 0]], rows[b], gsem[b])

        for win in range(AHEAD):
            start_gather(win)
        for win in range(WINS):
            b = win % RING
            gd.pop(win).wait()
            sd[win] = pltpu.async_copy(
                rows[b], acc.at[eall.at[win, 1]], ssem[b], add=True)
            nxt = win + AHEAD
            if nxt < WINS:
                prev = nxt - RING
                if prev >= 0:
                    sd.pop(prev).wait()
                start_gather(nxt)
        for win in sorted(sd):
            sd[win].wait()
        plsc.subcore_barrier()
        if merged:
            _drain_cols(acc, out_hbm, s, c, w)
        else:
            pltpu.sync_copy(acc.at[pl.ds(s * RPT, RPT)],
                            out_hbm.at[c, pl.ds(s * RPT, RPT)])

    return scat(hp, eidx)


WINS2 = EPAD // NS // KW
HW = 64


def _edge_scatter_colsplit(hp, eidx_cs):
    @functools.partial(
        pl.kernel,
        out_type=jax.ShapeDtypeStruct((N, NC * HW), jnp.float32),
        mesh=_mesh(),
        compiler_params=_SC_PARAMS,
        scratch_types=[
            [pltpu.VMEM((2, KW), jnp.int32)] * IR,
            [pltpu.VMEM((KW, HW), jnp.float32)] * RING,
            pltpu.VMEM((ZR, HW), jnp.float32),
            pltpu.VMEM_SHARED((N, HW), jnp.float32),
            pltpu.VMEM_SHARED((NPAD, HW), jnp.float32),
            [pltpu.SemaphoreType.DMA] * IR,
            [pltpu.SemaphoreType.DMA] * RING,
            [pltpu.SemaphoreType.DMA] * RING,
            pltpu.SemaphoreType.DMA,
            pltpu.SemaphoreType.DMA,
        ],
    )
    def scat(hp_hbm, eidx_hbm, out_hbm, ibuf, rows, zb, tbl, acc, isem,
             gsem, ssem, zsem, tsem):
        c = lax.axis_index("c")
        s = lax.axis_index("s")
        td = pltpu.async_copy(
            hp_hbm.at[pl.ds(s * CHK, CHK), pl.ds(c * HW, HW)],
            tbl.at[pl.ds(s * CHK, CHK)], tsem)
        idxd = {}
        gd = {}
        sd = {}

        def start_idx(win):
            idxd[win] = pltpu.async_copy(
                eidx_hbm.at[s, win], ibuf[win % IR], isem[win % IR])

        def start_gather(win):
            b = win % RING
            gd[win] = pltpu.async_copy(
                tbl.at[ibuf[win % IR].at[0]], rows[b], gsem[b])

        for win in range(4):
            start_idx(win)
        _zero_acc_2d(zb, acc, s, HW, zsem)
        td.wait()
        plsc.subcore_barrier()
        for win in range(AHEAD):
            idxd.pop(win).wait()
            start_gather(win)
        for win in range(WINS2):
            b = win % RING
            gd.pop(win).wait()
            sd[win] = pltpu.async_copy(
                rows[b], acc.at[ibuf[win % IR].at[1]], ssem[b], add=True)
            prev = win - (RING - AHEAD)
            if prev >= 0:
                sd.pop(prev).wait()
            if win + 4 < WINS2:
                start_idx(win + 4)
            nxt = win + AHEAD
            if nxt < WINS2:
                idxd.pop(nxt).wait()
                start_gather(nxt)
        for win in sorted(sd):
            sd[win].wait()
        plsc.subcore_barrier()
        _drain_cols(acc, out_hbm, s, c, HW)

    return scat(hp, eidx_cs)


def _edge_scatter(hp, eidx, w):
    return _edge_scatter_preload(hp, eidx, w)


_MM = dict(preferred_element_type=jnp.float32,
           precision=jax.lax.Precision.HIGHEST)


def _row_spec(width):
    return pl.BlockSpec((BN, width), lambda i: (i, 0))


def _part_spec(width):
    return pl.BlockSpec((NC, BN, width), lambda i: (0, i, 0))


def _full_spec(shape):
    nd = len(shape)
    return pl.BlockSpec(shape, lambda i: (0,) * nd)


def _dinv_body(dp_ref, o_ref):
    deg = dp_ref[0:80] + dp_ref[80:160] + 1.0
    o_ref[...] = lax.rsqrt(deg)


def _mm1_body(x_ref, w_ref, dv_ref, o_ref):
    o_ref[...] = dv_ref[...] * jnp.dot(x_ref[...], w_ref[...], **_MM)


def _epmm_body(s_ref, u_ref, dv_ref, b_ref, w_ref, o_ref):
    p = s_ref[...] + u_ref[...]
    h = jnp.maximum(dv_ref[...] * p + b_ref[...], 0.0)
    o_ref[...] = dv_ref[...] * jnp.dot(h, w_ref[...], **_MM)


def _mid_body(s_ref, u_ref, dv_ref, b_ref, wmu_ref, bmu_ref, wlv_ref,
              blv_ref, eps_ref, mu_ref, lv_ref, u3_ref):
    p = s_ref[:, :64] + s_ref[:, 64:] + u_ref[...]
    h2 = jnp.maximum(dv_ref[...] * p + b_ref[...], 0.0)
    mu = jnp.dot(h2, wmu_ref[...], **_MM) + bmu_ref[...]
    lv = jnp.dot(h2, wlv_ref[...], **_MM) + blv_ref[...]
    z = mu + lv * eps_ref[...]
    mu_ref[...] = mu
    lv_ref[...] = lv
    u3_ref[...] = dv_ref[...] * z


def _aggmm_body(s_ref, u_ref, dv_ref, w_ref, b_ref, o_ref):
    agg = dv_ref[...] * (s_ref[0] + s_ref[1] + u_ref[...])
    h = jnp.maximum(jnp.dot(agg, w_ref[...], **_MM) + b_ref[...], 0.0)
    o_ref[...] = dv_ref[...] * h


def _agg2mm_body(s_ref, u_ref, dv_ref, w4_ref, b4_ref, w5_ref, o_ref):
    agg = dv_ref[...] * (s_ref[:, :64] + s_ref[:, 64:] + u_ref[...])
    h4 = jnp.maximum(jnp.dot(agg, w4_ref[...], **_MM) + b4_ref[...], 0.0)
    o_ref[...] = dv_ref[...] * jnp.dot(h4, w5_ref[...], **_MM)


def _final_body(s_ref, u_ref, dv_ref, b_ref, o_ref):
    p = s_ref[...] + u_ref[...]
    o_ref[...] = jax.nn.sigmoid(dv_ref[...] * p + b_ref[...])


def kernel(x, W1, b1, W2, b2, Wmu, bmu, Wlv, blv, W3, b3, W4, b4, W5, b5,
           edge_index):
    f32 = jnp.float32
    src = edge_index[0]
    dst = edge_index[1]
    pad = EPAD - E
    padi = jnp.arange(pad, dtype=jnp.int32)
    srcf = jnp.concatenate([src, padi % N])
    dstf = jnp.concatenate([dst, N + padi % (NPAD - N)])
    srcp = srcf.reshape(NW, WINS, KW)
    dstp = dstf.reshape(NW, WINS, KW)
    eidx = jnp.stack([srcp, dstp], axis=2)
    eidx_cs = jnp.stack([srcf.reshape(NS, WINS2, KW),
                         dstf.reshape(NS, WINS2, KW)], axis=2)

    degp = _hist(eidx)
    dinv80 = pl.pallas_call(
        _dinv_body,
        out_shape=jax.ShapeDtypeStruct((80, 128), f32),
    )(degp.reshape(160, 128))
    dv = dinv80.reshape(NPAD, 1)[:N]

    b1r, b2r, b3r, b4r, b5r = (b.reshape(1, -1) for b in (b1, b2, b3, b4, b5))
    bmur, blvr = bmu.reshape(1, -1), blv.reshape(1, -1)
    eps = jax.random.normal(jax.random.key(1234), (N, Wmu.shape[1]), dtype=f32)

    dv_spec = pl.BlockSpec((BN, 1), lambda i: (i, 0))

    u1 = pl.pallas_call(
        _mm1_body,
        grid=(GRID,),
        in_specs=[_row_spec(128), _full_spec((128, 128)), dv_spec],
        out_specs=_row_spec(128),
        out_shape=jax.ShapeDtypeStruct((N, 128), f32),
    )(x, W1, dv)
    s1 = _edge_scatter_colsplit(u1, eidx_cs)

    u2 = pl.pallas_call(
        _epmm_body,
        grid=(GRID,),
        in_specs=[_row_spec(128), _row_spec(128), dv_spec,
                  _full_spec((1, 128)), _full_spec((128, 64))],
        out_specs=_row_spec(64),
        out_shape=jax.ShapeDtypeStruct((N, 64), f32),
    )(s1, u1, dv, b1r, W2)
    s2 = _edge_scatter(u2, eidx, 64)

    mu, lv, u3 = pl.pallas_call(
        _mid_body,
        grid=(GRID,),
        in_specs=[_row_spec(128), _row_spec(64), dv_spec, _full_spec((1, 64)),
                  _full_spec((64, 32)), _full_spec((1, 32)),
                  _full_spec((64, 32)), _full_spec((1, 32)), _row_spec(32)],
        out_specs=[_row_spec(32), _row_spec(32), _row_spec(32)],
        out_shape=(jax.ShapeDtypeStruct((N, 32), f32),
                   jax.ShapeDtypeStruct((N, 32), f32),
                   jax.ShapeDtypeStruct((N, 32), f32)),
    )(s2, u2, dv, b2r, Wmu, bmur, Wlv, blvr, eps)
    s3 = _edge_scatter(u3, eidx, 32)

    u4 = pl.pallas_call(
        _aggmm_body,
        grid=(GRID,),
        in_specs=[_part_spec(32), _row_spec(32), dv_spec,
                  _full_spec((32, 64)), _full_spec((1, 64))],
        out_specs=_row_spec(64),
        out_shape=jax.ShapeDtypeStruct((N, 64), f32),
    )(s3, u3, dv, W3, b3r)
    s4 = _edge_scatter(u4, eidx, 64)

    u5 = pl.pallas_call(
        _agg2mm_body,
        grid=(GRID,),
        in_specs=[_row_spec(128), _row_spec(64), dv_spec,
                  _full_spec((64, 128)), _full_spec((1, 128)),
                  _full_spec((128, 128))],
        out_specs=_row_spec(128),
        out_shape=jax.ShapeDtypeStruct((N, 128), f32),
    )(s4, u4, dv, W4, b4r, W5)
    s5 = _edge_scatter_colsplit(u5, eidx_cs)

    recon = pl.pallas_call(
        _final_body,
        grid=(GRID,),
        in_specs=[_row_spec(128), _row_spec(128), dv_spec,
                  _full_spec((1, 128))],
        out_specs=_row_spec(128),
        out_shape=jax.ShapeDtypeStruct((N, 128), f32),
    )(s5, u5, dv, b5r)
    return (recon, mu, lv)
